# baseline (speedup 1.0000x reference)
import functools

import jax
import jax.numpy as jnp
from jax import lax
from jax.experimental import pallas as pl
from jax.experimental.pallas import tpu as pltpu
from jax.experimental.pallas import tpu_sc as plsc

ROWS = 16384
COLS = 200
D = 32
VOCAB_ROWS = 53
LANES = 16
B_TOTAL = ROWS * COLS
NUM_CORES = 2
NUM_SUBCORES = 16
NW = NUM_CORES * NUM_SUBCORES
B_PER_W = B_TOTAL // NW
CHUNK = 1600
N_PAIRS = B_PER_W // (2 * CHUNK)
N_GROUPS = CHUNK // LANES


def _make_gather():
    mesh = plsc.VectorSubcoreMesh(core_axis_name="c", subcore_axis_name="s")

    @functools.partial(
        pl.kernel,
        mesh=mesh,
        out_type=jax.ShapeDtypeStruct((B_TOTAL * D,), jnp.float32),
        compiler_params=pltpu.CompilerParams(
            use_tc_tiling_on_sc=False, needs_layout_passes=False),
        scratch_types=[
            pltpu.VMEM((VOCAB_ROWS * D,), jnp.float32),
            pltpu.VMEM((CHUNK,), jnp.int32),
            pltpu.VMEM((CHUNK,), jnp.int32),
            pltpu.VMEM((CHUNK * D,), jnp.float32),
            pltpu.VMEM((CHUNK * D,), jnp.float32),
            pltpu.SemaphoreType.DMA,
            pltpu.SemaphoreType.DMA,
        ],
    )
    def gather_kernel(idx_hbm, table_hbm, out_hbm,
                      table_v, idx0_v, idx1_v, out0_v, out1_v,
                      sem_o0, sem_o1):
        sid = lax.axis_index("s")
        cid = lax.axis_index("c")
        wid = sid * NUM_CORES + cid
        base0 = wid * B_PER_W

        pltpu.sync_copy(table_hbm, table_v)
        lane_off = lax.iota(jnp.int32, LANES) * D

        def fill(idx_v, out_v):
            def group(g, carry):
                rows = idx_v[pl.ds(g * LANES, LANES)]
                ga = rows * D
                sa = g * (LANES * D) + lane_off
                for j in range(D):
                    vals = plsc.load_gather(table_v, [ga + j])
                    plsc.store_scatter(out_v, [sa + j], vals)
                return carry

            lax.fori_loop(0, N_GROUPS, group, 0)

        def body(j, carry):
            ba = base0 + (2 * j) * CHUNK
            bb = ba + CHUNK

            @pl.when(j > 0)
            def _():
                pltpu.make_async_copy(
                    out_hbm.at[pl.ds(ba * D, CHUNK * D)], out0_v, sem_o0).wait()
            pltpu.sync_copy(idx_hbm.at[pl.ds(ba, CHUNK)], idx0_v)
            fill(idx0_v, out0_v)
            pltpu.async_copy(out0_v, out_hbm.at[pl.ds(ba * D, CHUNK * D)],
                             sem_o0)

            @pl.when(j > 0)
            def _():
                pltpu.make_async_copy(
                    out_hbm.at[pl.ds(bb * D, CHUNK * D)], out1_v, sem_o1).wait()
            pltpu.sync_copy(idx_hbm.at[pl.ds(bb, CHUNK)], idx1_v)
            fill(idx1_v, out1_v)
            pltpu.async_copy(out1_v, out_hbm.at[pl.ds(bb * D, CHUNK * D)],
                             sem_o1)
            return carry

        lax.fori_loop(0, N_PAIRS, body, 0)

        pltpu.make_async_copy(out_hbm.at[pl.ds(base0 * D, CHUNK * D)], out0_v,
                              sem_o0).wait()
        pltpu.make_async_copy(out_hbm.at[pl.ds(base0 * D, CHUNK * D)], out1_v,
                              sem_o1).wait()

    return gather_kernel


_gather = _make_gather()


@jax.jit
def kernel(card_indices, embedding_table):
    flat_idx = card_indices.reshape(B_TOTAL).astype(jnp.int32)
    flat_table = embedding_table.reshape(VOCAB_ROWS * D)
    out = _gather(flat_idx, flat_table)
    return out.reshape(ROWS, COLS, D)


# speedup vs baseline: 2.3617x; 2.3617x over previous
import functools

import jax
import jax.numpy as jnp
from jax import lax
from jax.experimental import pallas as pl
from jax.experimental.pallas import tpu as pltpu
from jax.experimental.pallas import tpu_sc as plsc

ROWS = 16384
COLS = 200
D = 32
VOCAB_ROWS = 53
LANES = 16
B_TOTAL = ROWS * COLS
NUM_CORES = 2
NUM_SUBCORES = 16
NW = NUM_CORES * NUM_SUBCORES
B_PER_W = B_TOTAL // NW
CHUNK = 1600
N_PAIRS = B_PER_W // (2 * CHUNK)
N_GROUPS = CHUNK // LANES


def _make_gather():
    mesh = plsc.VectorSubcoreMesh(core_axis_name="c", subcore_axis_name="s")

    @functools.partial(
        pl.kernel,
        mesh=mesh,
        out_type=jax.ShapeDtypeStruct((B_TOTAL * D,), jnp.float32),
        compiler_params=pltpu.CompilerParams(
            use_tc_tiling_on_sc=False, needs_layout_passes=False),
        scratch_types=[
            pltpu.VMEM((VOCAB_ROWS * D,), jnp.float32),
            pltpu.VMEM((CHUNK,), jnp.int32),
            pltpu.VMEM((CHUNK,), jnp.int32),
            pltpu.VMEM((CHUNK * D,), jnp.float32),
            pltpu.VMEM((CHUNK * D,), jnp.float32),
            pltpu.SemaphoreType.DMA,
            pltpu.SemaphoreType.DMA,
        ],
    )
    def gather_kernel(idx_hbm, table_hbm, out_hbm,
                      table_v, idx0_v, idx1_v, out0_v, out1_v,
                      sem_o0, sem_o1):
        sid = lax.axis_index("s")
        cid = lax.axis_index("c")
        wid = sid * NUM_CORES + cid
        base0 = wid * B_PER_W

        pltpu.sync_copy(table_hbm, table_v)
        lane = lax.iota(jnp.int32, LANES)
        lane_off = lane * D

        def fill(idx_v, out_v):
            def group(g, carry):
                rows_a = idx_v[pl.ds((2 * g) * LANES, LANES)]
                rows_b = idx_v[pl.ds((2 * g + 1) * LANES, LANES)]
                ga_a = rows_a * D
                ga_b = rows_b * D
                sa_a = (2 * g) * (LANES * D) + lane_off
                sa_b = sa_a + LANES * D
                colv = lane
                for j in range(D):
                    va = plsc.load_gather(table_v, [ga_a + colv])
                    vb = plsc.load_gather(table_v, [ga_b + colv])
                    plsc.store_scatter(out_v, [sa_a + colv], va)
                    plsc.store_scatter(out_v, [sa_b + colv], vb)
                    colv = (colv + 1) & (D - 1)
                return carry

            lax.fori_loop(0, N_GROUPS // 2, group, 0)

        def body(j, carry):
            ba = base0 + (2 * j) * CHUNK
            bb = ba + CHUNK

            @pl.when(j > 0)
            def _():
                pltpu.make_async_copy(
                    out_hbm.at[pl.ds(ba * D, CHUNK * D)], out0_v, sem_o0).wait()
            pltpu.sync_copy(idx_hbm.at[pl.ds(ba, CHUNK)], idx0_v)
            fill(idx0_v, out0_v)
            pltpu.async_copy(out0_v, out_hbm.at[pl.ds(ba * D, CHUNK * D)],
                             sem_o0)

            @pl.when(j > 0)
            def _():
                pltpu.make_async_copy(
                    out_hbm.at[pl.ds(bb * D, CHUNK * D)], out1_v, sem_o1).wait()
            pltpu.sync_copy(idx_hbm.at[pl.ds(bb, CHUNK)], idx1_v)
            fill(idx1_v, out1_v)
            pltpu.async_copy(out1_v, out_hbm.at[pl.ds(bb * D, CHUNK * D)],
                             sem_o1)
            return carry

        lax.fori_loop(0, N_PAIRS, body, 0)

        pltpu.make_async_copy(out_hbm.at[pl.ds(base0 * D, CHUNK * D)], out0_v,
                              sem_o0).wait()
        pltpu.make_async_copy(out_hbm.at[pl.ds(base0 * D, CHUNK * D)], out1_v,
                              sem_o1).wait()

    return gather_kernel


_gather = _make_gather()


@jax.jit
def kernel(card_indices, embedding_table):
    flat_idx = card_indices.reshape(B_TOTAL).astype(jnp.int32)
    flat_table = embedding_table.reshape(VOCAB_ROWS * D)
    out = _gather(flat_idx, flat_table)
    return out.reshape(ROWS, COLS, D)


# speedup vs baseline: 6.9803x; 2.9557x over previous
import functools

import jax
import jax.numpy as jnp
from jax import lax
from jax.experimental import pallas as pl
from jax.experimental.pallas import tpu as pltpu
from jax.experimental.pallas import tpu_sc as plsc

ROWS = 16384
COLS = 200
D = 32
VOCAB_ROWS = 53
TABLE_WORDS = VOCAB_ROWS * D
REP_STRIDE = TABLE_WORDS + 1
LANES = 16
NUM_CORES = 2
NUM_SUBCORES = 16
NW = NUM_CORES * NUM_SUBCORES
R_W = ROWS // NW
C_BLK = 8
N_CBLK = COLS // C_BLK
R16 = R_W // LANES


def _make_gather():
    mesh = plsc.VectorSubcoreMesh(core_axis_name="c", subcore_axis_name="s")

    @functools.partial(
        pl.kernel,
        mesh=mesh,
        out_type=jax.ShapeDtypeStruct((COLS, D, ROWS), jnp.float32),
        compiler_params=pltpu.CompilerParams(
            use_tc_tiling_on_sc=True, needs_layout_passes=False),
        scratch_types=[
            pltpu.VMEM((LANES * REP_STRIDE,), jnp.float32),
            pltpu.VMEM((C_BLK, R_W), jnp.int32),
            pltpu.VMEM((2, D, R_W), jnp.float32),
            pltpu.VMEM((2, D, R_W), jnp.float32),
            pltpu.SemaphoreType.DMA,
            pltpu.SemaphoreType.DMA,
        ],
    )
    def gather_kernel(idx_hbm, table_hbm, out_hbm,
                      rep_v, idx_v, out0_v, out1_v, sem0, sem1):
        sid = lax.axis_index("s")
        cid = lax.axis_index("c")
        wid = sid * NUM_CORES + cid
        rw = wid * R_W
        lane = lax.iota(jnp.int32, LANES)

        pltpu.sync_copy(table_hbm, rep_v.at[pl.ds(0, TABLE_WORDS)])
        def rep_build(k, carry):
            src = rep_v[pl.ds(k * LANES, LANES)]
            for l in range(1, LANES):
                plsc.store_scatter(
                    rep_v, [l * REP_STRIDE + k * LANES + lane], src)
            return carry
        lax.fori_loop(0, TABLE_WORDS // LANES, rep_build, 0)

        rep_off = lane * REP_STRIDE

        bufs = (out0_v, out1_v)
        sems = (sem0, sem1)

        def fill(buf, pslot):
            def r_group(r16, carry):
                r0 = r16 * LANES
                for c2 in range(2):
                    rows = idx_v[2 * pslot + c2, pl.ds(r0, LANES)]
                    base = rep_off + rows * D
                    for j in range(D):
                        vals = plsc.load_gather(rep_v, [base + j])
                        buf[c2, j, pl.ds(r0, LANES)] = vals
                return carry
            lax.fori_loop(0, R16, r_group, 0)

        def body(cb, carry):
            c0 = cb * C_BLK
            pltpu.sync_copy(
                idx_hbm.at[pl.ds(c0, C_BLK), pl.ds(rw, R_W)], idx_v)
            for p in range(4):
                buf = bufs[p % 2]
                sem = sems[p % 2]
                if p < 2:
                    @pl.when(cb > 0)
                    def _():
                        pltpu.make_async_copy(
                            out_hbm.at[pl.ds(0, 2), pl.ds(0, D),
                                       pl.ds(rw, R_W)],
                            buf, sem).wait()
                else:
                    pltpu.make_async_copy(
                        out_hbm.at[pl.ds(0, 2), pl.ds(0, D), pl.ds(rw, R_W)],
                        buf, sem).wait()
                fill(buf, p)
                pltpu.async_copy(
                    buf,
                    out_hbm.at[pl.ds(c0 + 2 * p, 2), pl.ds(0, D),
                               pl.ds(rw, R_W)],
                    sem)
            return carry

        lax.fori_loop(0, N_CBLK, body, 0)

        for b in range(2):
            pltpu.make_async_copy(
                out_hbm.at[pl.ds(0, 2), pl.ds(0, D), pl.ds(rw, R_W)],
                bufs[b], sems[b]).wait()

    return gather_kernel


_gather = _make_gather()


@jax.jit
def kernel(card_indices, embedding_table):
    idx_t = card_indices.T.astype(jnp.int32)
    flat_table = embedding_table.reshape(TABLE_WORDS)
    out = _gather(idx_t, flat_table)
    return jnp.transpose(out, (2, 0, 1))


# speedup vs baseline: 15.6468x; 2.2416x over previous
import functools

import jax
import jax.numpy as jnp
from jax import lax
from jax.experimental import pallas as pl
from jax.experimental.pallas import tpu as pltpu
from jax.experimental.pallas import tpu_sc as plsc

ROWS = 16384
COLS = 200
D = 32
VOCAB_ROWS = 53
TABLE_WORDS = VOCAB_ROWS * D
REP_STRIDE = TABLE_WORDS + 1
LANES = 16
NUM_CORES = 2
NUM_SUBCORES = 16
NW = NUM_CORES * NUM_SUBCORES
R_W = ROWS // NW
C_BLK = 8
N_CBLK = COLS // C_BLK
R16 = R_W // LANES


def _make_gather():
    mesh = plsc.VectorSubcoreMesh(core_axis_name="c", subcore_axis_name="s")

    @functools.partial(
        pl.kernel,
        mesh=mesh,
        out_type=jax.ShapeDtypeStruct((COLS, D, ROWS), jnp.float32),
        compiler_params=pltpu.CompilerParams(
            use_tc_tiling_on_sc=True, needs_layout_passes=False),
        scratch_types=[
            pltpu.VMEM((LANES * REP_STRIDE,), jnp.float32),
            pltpu.VMEM((C_BLK, R_W), jnp.int32),
            pltpu.VMEM((2, D, R_W), jnp.float32),
            pltpu.VMEM((2, D, R_W), jnp.float32),
            pltpu.SemaphoreType.DMA,
            pltpu.SemaphoreType.DMA,
        ],
    )
    def gather_kernel(idx_hbm, table_hbm, out_hbm,
                      rep_v, idx_v, out0_v, out1_v, sem0, sem1):
        sid = lax.axis_index("s")
        cid = lax.axis_index("c")
        wid = sid * NUM_CORES + cid
        rw = wid * R_W
        lane = lax.iota(jnp.int32, LANES)

        pltpu.sync_copy(table_hbm, rep_v.at[pl.ds(0, TABLE_WORDS)])
        def rep_build(k, carry):
            src = rep_v[pl.ds(k * LANES, LANES)]
            for l in range(1, LANES):
                plsc.store_scatter(
                    rep_v, [l * REP_STRIDE + k * LANES + lane], src)
            return carry
        lax.fori_loop(0, TABLE_WORDS // LANES, rep_build, 0)

        rep_off = lane * REP_STRIDE

        bufs = (out0_v, out1_v)
        sems = (sem0, sem1)

        def fill(buf, pslot):
            def r_group(r16, carry):
                r0a = (2 * r16) * LANES
                r0b = r0a + LANES
                bases = []
                for c2 in range(2):
                    for r0 in (r0a, r0b):
                        rows = idx_v[2 * pslot + c2, pl.ds(r0, LANES)]
                        bases.append((c2, r0, rep_off + rows * D))
                for j in range(D):
                    vals = [plsc.load_gather(rep_v, [b + j])
                            for (_, _, b) in bases]
                    for (c2, r0, _), v in zip(bases, vals):
                        buf[c2, j, pl.ds(r0, LANES)] = v
                return carry
            lax.fori_loop(0, R16 // 2, r_group, 0)

        def body(cb, carry):
            c0 = cb * C_BLK
            pltpu.sync_copy(
                idx_hbm.at[pl.ds(c0, C_BLK), pl.ds(rw, R_W)], idx_v)
            for p in range(4):
                buf = bufs[p % 2]
                sem = sems[p % 2]
                if p < 2:
                    @pl.when(cb > 0)
                    def _():
                        pltpu.make_async_copy(
                            out_hbm.at[pl.ds(0, 2), pl.ds(0, D),
                                       pl.ds(rw, R_W)],
                            buf, sem).wait()
                else:
                    pltpu.make_async_copy(
                        out_hbm.at[pl.ds(0, 2), pl.ds(0, D), pl.ds(rw, R_W)],
                        buf, sem).wait()
                fill(buf, p)
                pltpu.async_copy(
                    buf,
                    out_hbm.at[pl.ds(c0 + 2 * p, 2), pl.ds(0, D),
                               pl.ds(rw, R_W)],
                    sem)
            return carry

        lax.fori_loop(0, N_CBLK, body, 0)

        for b in range(2):
            pltpu.make_async_copy(
                out_hbm.at[pl.ds(0, 2), pl.ds(0, D), pl.ds(rw, R_W)],
                bufs[b], sems[b]).wait()

    return gather_kernel


_gather = _make_gather()


@jax.jit
def kernel(card_indices, embedding_table):
    idx_t = card_indices.T.astype(jnp.int32)
    flat_table = embedding_table.reshape(TABLE_WORDS)
    out = _gather(idx_t, flat_table)
    return jnp.transpose(out, (2, 0, 1))


# speedup vs baseline: 19.0120x; 1.2151x over previous
import functools

import jax
import jax.numpy as jnp
from jax import lax
from jax.experimental import pallas as pl
from jax.experimental.pallas import tpu as pltpu
from jax.experimental.pallas import tpu_sc as plsc

ROWS = 16384
COLS = 200
D = 32
VOCAB_ROWS = 53
TABLE_WORDS = VOCAB_ROWS * D
REP_STRIDE = TABLE_WORDS + 1
LANES = 16
NUM_CORES = 2
NUM_SUBCORES = 16
NW = NUM_CORES * NUM_SUBCORES
R_W = ROWS // NW
C_BLK = 8
N_CBLK = COLS // C_BLK
R16 = R_W // LANES


def _make_gather():
    mesh = plsc.VectorSubcoreMesh(core_axis_name="c", subcore_axis_name="s")

    @functools.partial(
        pl.kernel,
        mesh=mesh,
        out_type=jax.ShapeDtypeStruct((COLS, D, ROWS), jnp.float32),
        compiler_params=pltpu.CompilerParams(
            use_tc_tiling_on_sc=True, needs_layout_passes=False),
        scratch_types=[
            pltpu.VMEM((LANES * REP_STRIDE,), jnp.float32),
            pltpu.VMEM((C_BLK, R_W), jnp.int32),
            pltpu.VMEM((2, D, R_W), jnp.float32),
            pltpu.VMEM((2, D, R_W), jnp.float32),
            pltpu.SemaphoreType.DMA,
            pltpu.SemaphoreType.DMA,
        ],
    )
    def gather_kernel(idx_hbm, table_hbm, out_hbm,
                      rep_v, idx_v, out0_v, out1_v, sem0, sem1):
        sid = lax.axis_index("s")
        cid = lax.axis_index("c")
        wid = sid * NUM_CORES + cid
        rw = wid * R_W
        lane = lax.iota(jnp.int32, LANES)

        pltpu.sync_copy(table_hbm, rep_v.at[pl.ds(0, TABLE_WORDS)])
        def rep_build(k, carry):
            src = rep_v[pl.ds(k * LANES, LANES)]
            for l in range(1, LANES):
                plsc.store_scatter(
                    rep_v, [l * REP_STRIDE + k * LANES + lane], src)
            return carry
        lax.fori_loop(0, TABLE_WORDS // LANES, rep_build, 0)

        rep_off = lane * REP_STRIDE

        bufs = (out0_v, out1_v)
        sems = (sem0, sem1)

        def fill(buf, pslot):
            def r_group(r16, carry):
                r0s = [(4 * r16 + k) * LANES for k in range(4)]
                bases = []
                for c2 in range(2):
                    for r0 in r0s:
                        rows = idx_v[2 * pslot + c2, pl.ds(r0, LANES)]
                        bases.append((c2, r0, rep_off + rows * D))
                for j in range(D):
                    vals = [plsc.load_gather(rep_v, [b + j])
                            for (_, _, b) in bases]
                    for (c2, r0, _), v in zip(bases, vals):
                        buf[c2, j, pl.ds(r0, LANES)] = v
                return carry
            lax.fori_loop(0, R16 // 4, r_group, 0)

        def body(cb, carry):
            c0 = cb * C_BLK
            pltpu.sync_copy(
                idx_hbm.at[pl.ds(c0, C_BLK), pl.ds(rw, R_W)], idx_v)
            for p in range(4):
                buf = bufs[p % 2]
                sem = sems[p % 2]
                if p < 2:
                    @pl.when(cb > 0)
                    def _():
                        pltpu.make_async_copy(
                            out_hbm.at[pl.ds(0, 2), pl.ds(0, D),
                                       pl.ds(rw, R_W)],
                            buf, sem).wait()
                else:
                    pltpu.make_async_copy(
                        out_hbm.at[pl.ds(0, 2), pl.ds(0, D), pl.ds(rw, R_W)],
                        buf, sem).wait()
                fill(buf, p)
                pltpu.async_copy(
                    buf,
                    out_hbm.at[pl.ds(c0 + 2 * p, 2), pl.ds(0, D),
                               pl.ds(rw, R_W)],
                    sem)
            return carry

        lax.fori_loop(0, N_CBLK, body, 0)

        for b in range(2):
            pltpu.make_async_copy(
                out_hbm.at[pl.ds(0, 2), pl.ds(0, D), pl.ds(rw, R_W)],
                bufs[b], sems[b]).wait()

    return gather_kernel


_gather = _make_gather()


@jax.jit
def kernel(card_indices, embedding_table):
    idx_t = card_indices.T.astype(jnp.int32)
    flat_table = embedding_table.reshape(TABLE_WORDS)
    out = _gather(idx_t, flat_table)
    return jnp.transpose(out, (2, 0, 1))


# speedup vs baseline: 23.2671x; 1.2238x over previous
import functools

import jax
import jax.numpy as jnp
from jax import lax
from jax.experimental import pallas as pl
from jax.experimental.pallas import tpu as pltpu
from jax.experimental.pallas import tpu_sc as plsc

ROWS = 16384
COLS = 200
D = 32
VOCAB_ROWS = 53
TABLE_WORDS = VOCAB_ROWS * D
REP_STRIDE = TABLE_WORDS + 1
LANES = 16
NUM_CORES = 2
NUM_SUBCORES = 16
NW = NUM_CORES * NUM_SUBCORES
R_W = ROWS // NW
C_BLK = 8
N_CBLK = COLS // C_BLK
R16 = R_W // LANES


def _make_gather():
    mesh = plsc.VectorSubcoreMesh(core_axis_name="c", subcore_axis_name="s")

    @functools.partial(
        pl.kernel,
        mesh=mesh,
        out_type=jax.ShapeDtypeStruct((COLS, D, ROWS), jnp.float32),
        compiler_params=pltpu.CompilerParams(
            use_tc_tiling_on_sc=True, needs_layout_passes=False),
        scratch_types=[
            pltpu.VMEM((LANES * REP_STRIDE,), jnp.float32),
            pltpu.VMEM((C_BLK, R_W), jnp.int32),
            pltpu.VMEM((2, D, R_W), jnp.float32),
            pltpu.VMEM((2, D, R_W), jnp.float32),
            pltpu.SemaphoreType.DMA,
            pltpu.SemaphoreType.DMA,
        ],
    )
    def gather_kernel(idx_hbm, table_hbm, out_hbm,
                      rep_v, idx_v, out0_v, out1_v, sem0, sem1):
        sid = lax.axis_index("s")
        cid = lax.axis_index("c")
        wid = sid * NUM_CORES + cid
        rw = wid * R_W
        lane = lax.iota(jnp.int32, LANES)

        pltpu.sync_copy(table_hbm, rep_v.at[pl.ds(0, TABLE_WORDS)])
        def rep_build(k, carry):
            src = rep_v[pl.ds(k * LANES, LANES)]
            for l in range(1, LANES):
                plsc.store_scatter(
                    rep_v, [l * REP_STRIDE + k * LANES + lane], src)
            return carry
        lax.fori_loop(0, TABLE_WORDS // LANES, rep_build, 0)

        rep_off = lane * REP_STRIDE

        bufs = (out0_v, out1_v)
        sems = (sem0, sem1)

        def fill(buf, pslot):
            @plsc.parallel_loop(0, R16 // 4, step=1)
            def r_group(r16):
                r0s = [(4 * r16 + k) * LANES for k in range(4)]
                bases = []
                for c2 in range(2):
                    for r0 in r0s:
                        rows = idx_v[2 * pslot + c2, pl.ds(r0, LANES)]
                        bases.append((c2, r0, rep_off + rows * D))
                vals_prev = None
                for j in range(D):
                    vals = []
                    for k, (c2, r0, b) in enumerate(bases):
                        vals.append(plsc.load_gather(rep_v, [b + j]))
                        if vals_prev is not None:
                            buf[c2, j - 1, pl.ds(r0, LANES)] = vals_prev[k]
                    vals_prev = vals
                for (c2, r0, _), v in zip(bases, vals_prev):
                    buf[c2, D - 1, pl.ds(r0, LANES)] = v

        def body(cb, carry):
            c0 = cb * C_BLK
            pltpu.sync_copy(
                idx_hbm.at[pl.ds(c0, C_BLK), pl.ds(rw, R_W)], idx_v)
            for p in range(4):
                buf = bufs[p % 2]
                sem = sems[p % 2]
                if p < 2:
                    @pl.when(cb > 0)
                    def _():
                        pltpu.make_async_copy(
                            out_hbm.at[pl.ds(0, 2), pl.ds(0, D),
                                       pl.ds(rw, R_W)],
                            buf, sem).wait()
                else:
                    pltpu.make_async_copy(
                        out_hbm.at[pl.ds(0, 2), pl.ds(0, D), pl.ds(rw, R_W)],
                        buf, sem).wait()
                fill(buf, p)
                pltpu.async_copy(
                    buf,
                    out_hbm.at[pl.ds(c0 + 2 * p, 2), pl.ds(0, D),
                               pl.ds(rw, R_W)],
                    sem)
            return carry

        lax.fori_loop(0, N_CBLK, body, 0)

        for b in range(2):
            pltpu.make_async_copy(
                out_hbm.at[pl.ds(0, 2), pl.ds(0, D), pl.ds(rw, R_W)],
                bufs[b], sems[b]).wait()

    return gather_kernel


_gather = _make_gather()


@jax.jit
def kernel(card_indices, embedding_table):
    idx_t = card_indices.T.astype(jnp.int32)
    flat_table = embedding_table.reshape(TABLE_WORDS)
    out = _gather(idx_t, flat_table)
    return jnp.transpose(out, (2, 0, 1))


# speedup vs baseline: 23.5794x; 1.0134x over previous
import functools

import jax
import jax.numpy as jnp
from jax import lax
from jax.experimental import pallas as pl
from jax.experimental.pallas import tpu as pltpu
from jax.experimental.pallas import tpu_sc as plsc

ROWS = 16384
COLS = 200
D = 32
VOCAB_ROWS = 53
TABLE_WORDS = VOCAB_ROWS * D
REP_STRIDE = TABLE_WORDS + 1
LANES = 16
NUM_CORES = 2
NUM_SUBCORES = 16
NW = NUM_CORES * NUM_SUBCORES
R_W = ROWS // NW
C_BLK = 8
N_CBLK = COLS // C_BLK
R16 = R_W // LANES


def _make_gather():
    mesh = plsc.VectorSubcoreMesh(core_axis_name="c", subcore_axis_name="s")

    @functools.partial(
        pl.kernel,
        mesh=mesh,
        out_type=jax.ShapeDtypeStruct((COLS, D, ROWS), jnp.float32),
        compiler_params=pltpu.CompilerParams(
            use_tc_tiling_on_sc=True, needs_layout_passes=False),
        scratch_types=[
            pltpu.VMEM((LANES * REP_STRIDE,), jnp.float32),
            pltpu.VMEM((C_BLK, R_W), jnp.int32),
            pltpu.VMEM((C_BLK, R_W), jnp.int32),
            pltpu.VMEM((2, D, R_W), jnp.float32),
            pltpu.VMEM((2, D, R_W), jnp.float32),
            pltpu.SemaphoreType.DMA,
            pltpu.SemaphoreType.DMA,
            pltpu.SemaphoreType.DMA,
            pltpu.SemaphoreType.DMA,
        ],
    )
    def gather_kernel(idx_hbm, table_hbm, out_hbm,
                      rep_v, idx0_v, idx1_v, out0_v, out1_v,
                      sem0, sem1, sem_i0, sem_i1):
        sid = lax.axis_index("s")
        cid = lax.axis_index("c")
        wid = sid * NUM_CORES + cid
        rw = wid * R_W
        lane = lax.iota(jnp.int32, LANES)

        pltpu.sync_copy(table_hbm, rep_v.at[pl.ds(0, TABLE_WORDS)])
        def rep_build(k, carry):
            src = rep_v[pl.ds(k * LANES, LANES)]
            for l in range(1, LANES):
                plsc.store_scatter(
                    rep_v, [l * REP_STRIDE + k * LANES + lane], src)
            return carry
        lax.fori_loop(0, TABLE_WORDS // LANES, rep_build, 0)

        rep_off = lane * REP_STRIDE

        bufs = (out0_v, out1_v)
        sems = (sem0, sem1)

        def fill(buf, pslot, idx_v):
            @plsc.parallel_loop(0, R16 // 4, step=1)
            def r_group(r16):
                r0s = [(4 * r16 + k) * LANES for k in range(4)]
                bases = []
                for c2 in range(2):
                    for r0 in r0s:
                        rows = idx_v[2 * pslot + c2, pl.ds(r0, LANES)]
                        bases.append((c2, r0, rep_off + rows * D))
                vals_prev = None
                for j in range(D):
                    vals = []
                    for k, (c2, r0, b) in enumerate(bases):
                        vals.append(plsc.load_gather(rep_v, [b + j]))
                        if vals_prev is not None:
                            buf[c2, j - 1, pl.ds(r0, LANES)] = vals_prev[k]
                    vals_prev = vals
                for (c2, r0, _), v in zip(bases, vals_prev):
                    buf[c2, D - 1, pl.ds(r0, LANES)] = v

        def idx_fetch(cb, idx_v, sem_i):
            pltpu.async_copy(
                idx_hbm.at[pl.ds(cb * C_BLK, C_BLK), pl.ds(rw, R_W)],
                idx_v, sem_i)

        def idx_wait(idx_v, sem_i):
            pltpu.make_async_copy(
                idx_hbm.at[pl.ds(0, C_BLK), pl.ds(rw, R_W)],
                idx_v, sem_i).wait()

        def block(cb, idx_v, first):
            c0 = cb * C_BLK
            for p in range(4):
                buf = bufs[p % 2]
                sem = sems[p % 2]
                if p < 2:
                    @pl.when(jnp.logical_not(first))
                    def _():
                        pltpu.make_async_copy(
                            out_hbm.at[pl.ds(0, 2), pl.ds(0, D),
                                       pl.ds(rw, R_W)],
                            buf, sem).wait()
                else:
                    pltpu.make_async_copy(
                        out_hbm.at[pl.ds(0, 2), pl.ds(0, D), pl.ds(rw, R_W)],
                        buf, sem).wait()
                fill(buf, p, idx_v)
                pltpu.async_copy(
                    buf,
                    out_hbm.at[pl.ds(c0 + 2 * p, 2), pl.ds(0, D),
                               pl.ds(rw, R_W)],
                    sem)

        idx_fetch(0, idx0_v, sem_i0)

        def body(j, carry):
            cb = 2 * j
            idx_wait(idx0_v, sem_i0)
            idx_fetch(cb + 1, idx1_v, sem_i1)
            block(cb, idx0_v, j == 0)
            idx_wait(idx1_v, sem_i1)
            idx_fetch(cb + 2, idx0_v, sem_i0)
            block(cb + 1, idx1_v, jnp.bool_(False))
            return carry

        lax.fori_loop(0, (N_CBLK - 1) // 2, body, 0)

        idx_wait(idx0_v, sem_i0)
        block(N_CBLK - 1, idx0_v, jnp.bool_(False))

        for b in range(2):
            pltpu.make_async_copy(
                out_hbm.at[pl.ds(0, 2), pl.ds(0, D), pl.ds(rw, R_W)],
                bufs[b], sems[b]).wait()

    return gather_kernel


_gather = _make_gather()


@jax.jit
def kernel(card_indices, embedding_table):
    idx_t = card_indices.T.astype(jnp.int32)
    flat_table = embedding_table.reshape(TABLE_WORDS)
    out = _gather(idx_t, flat_table)
    return jnp.transpose(out, (2, 0, 1))
